# initial kernel scaffold (unmeasured)
import jax
import jax.numpy as jnp
from jax import lax
from jax.experimental import pallas as pl
from jax.experimental.pallas import tpu as pltpu


def kernel(
    x,
):
    def body(*refs):
        pass

    out_shape = jax.ShapeDtypeStruct(..., jnp.float32)
    return pl.pallas_call(body, out_shape=out_shape)(...)


# baseline (device time: 10515 ns/iter reference)
import jax
import jax.numpy as jnp
from jax import lax
from jax.experimental import pallas as pl
from jax.experimental.pallas import tpu as pltpu


def kernel(x):
    m, n = x.shape[2], x.shape[3]
    x = x.reshape(m, n).astype(jnp.bfloat16)

    def body(x_ref, out_ref, comm_ref, send_sems, recv_sems):
        my_x = lax.axis_index("x")
        my_y = lax.axis_index("y")
        x_nbr = (1 - my_x, my_y)
        y_nbr = (my_x, 1 - my_y)

        barrier_sem = pltpu.get_barrier_semaphore()
        for nbr in (x_nbr, y_nbr):
            pl.semaphore_signal(
                barrier_sem, inc=1,
                device_id=nbr, device_id_type=pl.DeviceIdType.MESH,
            )
        pl.semaphore_wait(barrier_sem, 2)

        rdma1 = pltpu.make_async_remote_copy(
            src_ref=x_ref,
            dst_ref=comm_ref.at[0],
            send_sem=send_sems.at[0],
            recv_sem=recv_sems.at[0],
            device_id=x_nbr,
            device_id_type=pl.DeviceIdType.MESH,
        )
        rdma1.start()
        rdma1.wait()
        out_ref[...] = x_ref[...] + comm_ref[0]

        rdma2 = pltpu.make_async_remote_copy(
            src_ref=out_ref,
            dst_ref=comm_ref.at[1],
            send_sem=send_sems.at[1],
            recv_sem=recv_sems.at[1],
            device_id=y_nbr,
            device_id_type=pl.DeviceIdType.MESH,
        )
        rdma2.start()
        rdma2.wait()
        out_ref[...] = out_ref[...] + comm_ref[1]

    return pl.pallas_call(
        body,
        out_shape=jax.ShapeDtypeStruct((m, n), jnp.bfloat16),
        in_specs=[pl.BlockSpec(memory_space=pltpu.VMEM)],
        out_specs=pl.BlockSpec(memory_space=pltpu.VMEM),
        scratch_shapes=[
            pltpu.VMEM((2, m, n), jnp.bfloat16),
            pltpu.SemaphoreType.DMA((2,)),
            pltpu.SemaphoreType.DMA((2,)),
        ],
        compiler_params=pltpu.CompilerParams(collective_id=0),
    )(x)


# device time: 9539 ns/iter; 1.1023x vs baseline; 1.1023x over previous
import jax
import jax.numpy as jnp
from jax import lax
from jax.experimental import pallas as pl
from jax.experimental.pallas import tpu as pltpu

N_CHUNKS = 4


def kernel(x):
    m, n = x.shape[2], x.shape[3]
    rows = m // N_CHUNKS

    def body(x_ref, out_ref, xb_ref, comm_ref, send_sems, recv_sems):
        my_x = lax.axis_index("x")
        my_y = lax.axis_index("y")
        x_nbr = (1 - my_x, my_y)
        y_nbr = (my_x, 1 - my_y)

        xb_ref[...] = x_ref[0, 0].astype(jnp.bfloat16)

        barrier_sem = pltpu.get_barrier_semaphore()
        for nbr in (x_nbr, y_nbr):
            pl.semaphore_signal(
                barrier_sem, inc=1,
                device_id=nbr, device_id_type=pl.DeviceIdType.MESH,
            )
        pl.semaphore_wait(barrier_sem, 2)

        def chunk(ref, c, slot=None):
            sl = pl.ds(c * rows, rows)
            return ref.at[slot, sl] if slot is not None else ref.at[sl]

        rdma1 = []
        for c in range(N_CHUNKS):
            r = pltpu.make_async_remote_copy(
                src_ref=chunk(xb_ref, c),
                dst_ref=chunk(comm_ref, c, 0),
                send_sem=send_sems.at[0, c],
                recv_sem=recv_sems.at[0, c],
                device_id=x_nbr,
                device_id_type=pl.DeviceIdType.MESH,
            )
            r.start()
            rdma1.append(r)

        rdma2 = []
        for c in range(N_CHUNKS):
            rdma1[c].wait_recv()
            sl = pl.ds(c * rows, rows)
            out_ref[sl, :] = xb_ref[sl, :] + comm_ref[0, sl, :]
            r = pltpu.make_async_remote_copy(
                src_ref=chunk(out_ref, c),
                dst_ref=chunk(comm_ref, c, 1),
                send_sem=send_sems.at[1, c],
                recv_sem=recv_sems.at[1, c],
                device_id=y_nbr,
                device_id_type=pl.DeviceIdType.MESH,
            )
            r.start()
            rdma2.append(r)

        for c in range(N_CHUNKS):
            rdma2[c].wait_send()
            rdma2[c].wait_recv()
            sl = pl.ds(c * rows, rows)
            out_ref[sl, :] = out_ref[sl, :] + comm_ref[1, sl, :]

        for c in range(N_CHUNKS):
            rdma1[c].wait_send()

    return pl.pallas_call(
        body,
        out_shape=jax.ShapeDtypeStruct((m, n), jnp.bfloat16),
        in_specs=[pl.BlockSpec(memory_space=pltpu.VMEM)],
        out_specs=pl.BlockSpec(memory_space=pltpu.VMEM),
        scratch_shapes=[
            pltpu.VMEM((m, n), jnp.bfloat16),
            pltpu.VMEM((2, m, n), jnp.bfloat16),
            pltpu.SemaphoreType.DMA((2, N_CHUNKS)),
            pltpu.SemaphoreType.DMA((2, N_CHUNKS)),
        ],
        compiler_params=pltpu.CompilerParams(collective_id=0),
    )(x)


# device time: 9533 ns/iter; 1.1030x vs baseline; 1.0006x over previous
import os

import jax
import jax.numpy as jnp
from jax import lax
from jax.experimental import pallas as pl
from jax.experimental.pallas import tpu as pltpu

N_CHUNKS = int(os.environ.get("KCHUNKS", "4"))


def kernel(x):
    m, n = x.shape[2], x.shape[3]
    rows = m // N_CHUNKS

    def body(x_ref, out_ref, xb_ref, comm_ref, send_sems, recv_sems):
        my_x = lax.axis_index("x")
        my_y = lax.axis_index("y")
        x_nbr = (1 - my_x, my_y)
        y_nbr = (my_x, 1 - my_y)

        xb_ref[...] = x_ref[0, 0].astype(jnp.bfloat16)

        barrier_sem = pltpu.get_barrier_semaphore()
        for nbr in (x_nbr, y_nbr):
            pl.semaphore_signal(
                barrier_sem, inc=1,
                device_id=nbr, device_id_type=pl.DeviceIdType.MESH,
            )
        pl.semaphore_wait(barrier_sem, 2)

        def chunk(ref, c, slot=None):
            sl = pl.ds(c * rows, rows)
            return ref.at[slot, sl] if slot is not None else ref.at[sl]

        rdma1 = []
        for c in range(N_CHUNKS):
            r = pltpu.make_async_remote_copy(
                src_ref=chunk(xb_ref, c),
                dst_ref=chunk(comm_ref, c, 0),
                send_sem=send_sems.at[0, c],
                recv_sem=recv_sems.at[0, c],
                device_id=x_nbr,
                device_id_type=pl.DeviceIdType.MESH,
            )
            r.start()
            rdma1.append(r)

        rdma2 = []
        for c in range(N_CHUNKS):
            rdma1[c].wait_recv()
            sl = pl.ds(c * rows, rows)
            out_ref[sl, :] = xb_ref[sl, :] + comm_ref[0, sl, :]
            r = pltpu.make_async_remote_copy(
                src_ref=chunk(out_ref, c),
                dst_ref=chunk(comm_ref, c, 1),
                send_sem=send_sems.at[1, c],
                recv_sem=recv_sems.at[1, c],
                device_id=y_nbr,
                device_id_type=pl.DeviceIdType.MESH,
            )
            r.start()
            rdma2.append(r)

        for c in range(N_CHUNKS):
            rdma2[c].wait_send()
            rdma2[c].wait_recv()
            sl = pl.ds(c * rows, rows)
            out_ref[sl, :] = out_ref[sl, :] + comm_ref[1, sl, :]

        for c in range(N_CHUNKS):
            rdma1[c].wait_send()

    return pl.pallas_call(
        body,
        out_shape=jax.ShapeDtypeStruct((m, n), jnp.bfloat16),
        in_specs=[pl.BlockSpec(memory_space=pltpu.VMEM)],
        out_specs=pl.BlockSpec(memory_space=pltpu.VMEM),
        scratch_shapes=[
            pltpu.VMEM((m, n), jnp.bfloat16),
            pltpu.VMEM((2, m, n), jnp.bfloat16),
            pltpu.SemaphoreType.DMA((2, N_CHUNKS)),
            pltpu.SemaphoreType.DMA((2, N_CHUNKS)),
        ],
        compiler_params=pltpu.CompilerParams(collective_id=0),
    )(x)


# device time: 9266 ns/iter; 1.1348x vs baseline; 1.0288x over previous
import os

import jax
import jax.numpy as jnp
from jax import lax
from jax.experimental import pallas as pl
from jax.experimental.pallas import tpu as pltpu

N_CHUNKS = int(os.environ.get("KCHUNKS", "8"))


def kernel(x):
    m, n = x.shape[2], x.shape[3]
    rows = m // N_CHUNKS

    def body(x_hbm, out_ref, xf_ref, xb_ref, comm_ref,
             in_sem, send_sems, recv_sems):
        my_x = lax.axis_index("x")
        my_y = lax.axis_index("y")
        x_nbr = (1 - my_x, my_y)
        y_nbr = (my_x, 1 - my_y)

        in_copy = pltpu.make_async_copy(x_hbm.at[0, 0], xf_ref, in_sem)
        in_copy.start()

        barrier_sem = pltpu.get_barrier_semaphore()
        for nbr in (x_nbr, y_nbr):
            pl.semaphore_signal(
                barrier_sem, inc=1,
                device_id=nbr, device_id_type=pl.DeviceIdType.MESH,
            )
        pl.semaphore_wait(barrier_sem, 2)

        in_copy.wait()
        xb_ref[...] = xf_ref[...].astype(jnp.bfloat16)

        def chunk(ref, c, slot=None):
            sl = pl.ds(c * rows, rows)
            return ref.at[slot, sl] if slot is not None else ref.at[sl]

        rdma1 = []
        for c in range(N_CHUNKS):
            r = pltpu.make_async_remote_copy(
                src_ref=chunk(xb_ref, c),
                dst_ref=chunk(comm_ref, c, 0),
                send_sem=send_sems.at[0, c],
                recv_sem=recv_sems.at[0, c],
                device_id=x_nbr,
                device_id_type=pl.DeviceIdType.MESH,
            )
            r.start()
            rdma1.append(r)

        rdma2 = []
        for c in range(N_CHUNKS):
            rdma1[c].wait_recv()
            sl = pl.ds(c * rows, rows)
            out_ref[sl, :] = xb_ref[sl, :] + comm_ref[0, sl, :]
            r = pltpu.make_async_remote_copy(
                src_ref=chunk(out_ref, c),
                dst_ref=chunk(comm_ref, c, 1),
                send_sem=send_sems.at[1, c],
                recv_sem=recv_sems.at[1, c],
                device_id=y_nbr,
                device_id_type=pl.DeviceIdType.MESH,
            )
            r.start()
            rdma2.append(r)

        for c in range(N_CHUNKS):
            rdma2[c].wait_send()
            rdma2[c].wait_recv()
            sl = pl.ds(c * rows, rows)
            out_ref[sl, :] = out_ref[sl, :] + comm_ref[1, sl, :]

        for c in range(N_CHUNKS):
            rdma1[c].wait_send()

    return pl.pallas_call(
        body,
        out_shape=jax.ShapeDtypeStruct((m, n), jnp.bfloat16),
        in_specs=[pl.BlockSpec(memory_space=pltpu.MemorySpace.HBM)],
        out_specs=pl.BlockSpec(memory_space=pltpu.VMEM),
        scratch_shapes=[
            pltpu.VMEM((m, n), jnp.float32),
            pltpu.VMEM((m, n), jnp.bfloat16),
            pltpu.VMEM((2, m, n), jnp.bfloat16),
            pltpu.SemaphoreType.DMA,
            pltpu.SemaphoreType.DMA((2, N_CHUNKS)),
            pltpu.SemaphoreType.DMA((2, N_CHUNKS)),
        ],
        compiler_params=pltpu.CompilerParams(collective_id=0),
    )(pltpu.with_memory_space_constraint(x, pltpu.MemorySpace.HBM))


# device time: 9155 ns/iter; 1.1486x vs baseline; 1.0121x over previous
import os

import jax
import jax.numpy as jnp
from jax import lax
from jax.experimental import pallas as pl
from jax.experimental.pallas import tpu as pltpu

N_CHUNKS = int(os.environ.get("KCHUNKS", "8"))


def kernel(x):
    m, n = x.shape[2], x.shape[3]
    rows = m // N_CHUNKS

    def body(x_hbm, out_ref, xf_ref, xb_ref, comm_ref,
             in_sems, send_sems, recv_sems):
        my_x = lax.axis_index("x")
        my_y = lax.axis_index("y")
        x_nbr = (1 - my_x, my_y)
        y_nbr = (my_x, 1 - my_y)

        def chunk(ref, c, slot=None):
            sl = pl.ds(c * rows, rows)
            return ref.at[slot, sl] if slot is not None else ref.at[sl]

        in_copies = []
        for c in range(N_CHUNKS):
            ic = pltpu.make_async_copy(
                chunk(x_hbm.at[0, 0], c), chunk(xf_ref, c), in_sems.at[c]
            )
            ic.start()
            in_copies.append(ic)

        barrier_sem = pltpu.get_barrier_semaphore()
        for nbr in (x_nbr, y_nbr):
            pl.semaphore_signal(
                barrier_sem, inc=1,
                device_id=nbr, device_id_type=pl.DeviceIdType.MESH,
            )
        pl.semaphore_wait(barrier_sem, 2)

        rdma1 = []
        for c in range(N_CHUNKS):
            in_copies[c].wait()
            sl = pl.ds(c * rows, rows)
            xb_ref[sl, :] = xf_ref[sl, :].astype(jnp.bfloat16)
            r = pltpu.make_async_remote_copy(
                src_ref=chunk(xb_ref, c),
                dst_ref=chunk(comm_ref, c, 0),
                send_sem=send_sems.at[0, c],
                recv_sem=recv_sems.at[0, c],
                device_id=x_nbr,
                device_id_type=pl.DeviceIdType.MESH,
            )
            r.start()
            rdma1.append(r)

        rdma2 = []
        for c in range(N_CHUNKS):
            rdma1[c].wait_recv()
            sl = pl.ds(c * rows, rows)
            out_ref[sl, :] = xb_ref[sl, :] + comm_ref[0, sl, :]
            r = pltpu.make_async_remote_copy(
                src_ref=chunk(out_ref, c),
                dst_ref=chunk(comm_ref, c, 1),
                send_sem=send_sems.at[1, c],
                recv_sem=recv_sems.at[1, c],
                device_id=y_nbr,
                device_id_type=pl.DeviceIdType.MESH,
            )
            r.start()
            rdma2.append(r)

        for c in range(N_CHUNKS):
            rdma2[c].wait_send()
            rdma2[c].wait_recv()
            sl = pl.ds(c * rows, rows)
            out_ref[sl, :] = out_ref[sl, :] + comm_ref[1, sl, :]

        for c in range(N_CHUNKS):
            rdma1[c].wait_send()

    return pl.pallas_call(
        body,
        out_shape=jax.ShapeDtypeStruct((m, n), jnp.bfloat16),
        in_specs=[pl.BlockSpec(memory_space=pltpu.MemorySpace.HBM)],
        out_specs=pl.BlockSpec(memory_space=pltpu.VMEM),
        scratch_shapes=[
            pltpu.VMEM((m, n), jnp.float32),
            pltpu.VMEM((m, n), jnp.bfloat16),
            pltpu.VMEM((2, m, n), jnp.bfloat16),
            pltpu.SemaphoreType.DMA((N_CHUNKS,)),
            pltpu.SemaphoreType.DMA((2, N_CHUNKS)),
            pltpu.SemaphoreType.DMA((2, N_CHUNKS)),
        ],
        compiler_params=pltpu.CompilerParams(collective_id=0),
    )(pltpu.with_memory_space_constraint(x, pltpu.MemorySpace.HBM))


# device time: 8771 ns/iter; 1.1988x vs baseline; 1.0438x over previous
import os

import jax
import jax.numpy as jnp
from jax import lax
from jax.experimental import pallas as pl
from jax.experimental.pallas import tpu as pltpu

N_CHUNKS = int(os.environ.get("KCHUNKS", "4"))


def kernel(x):
    m, n = x.shape[2], x.shape[3]
    half = m // 2
    rows = half // N_CHUNKS

    def body(x_hbm, out_ref, xf_ref, xb_ref, comm_ref,
             in_sems, send_sems, recv_sems):
        my_x = lax.axis_index("x")
        my_y = lax.axis_index("y")
        x_nbr = (1 - my_x, my_y)
        y_nbr = (my_x, 1 - my_y)
        partners = ((x_nbr, y_nbr), (y_nbr, x_nbr))

        def sl(p, c):
            return pl.ds(p * half + c * rows, rows)

        in_copies = {}
        for c in range(N_CHUNKS):
            for p in range(2):
                ic = pltpu.make_async_copy(
                    x_hbm.at[0, 0, sl(p, c)], xf_ref.at[sl(p, c)],
                    in_sems.at[p, c],
                )
                ic.start()
                in_copies[p, c] = ic

        barrier_sem = pltpu.get_barrier_semaphore()
        for nbr in (x_nbr, y_nbr):
            pl.semaphore_signal(
                barrier_sem, inc=1,
                device_id=nbr, device_id_type=pl.DeviceIdType.MESH,
            )
        pl.semaphore_wait(barrier_sem, 2)

        rdma1 = {}
        for c in range(N_CHUNKS):
            for p in range(2):
                in_copies[p, c].wait()
                s = sl(p, c)
                xb_ref[s, :] = xf_ref[s, :].astype(jnp.bfloat16)
                r = pltpu.make_async_remote_copy(
                    src_ref=xb_ref.at[s],
                    dst_ref=comm_ref.at[0, s],
                    send_sem=send_sems.at[0, p, c],
                    recv_sem=recv_sems.at[0, p, c],
                    device_id=partners[p][0],
                    device_id_type=pl.DeviceIdType.MESH,
                )
                r.start()
                rdma1[p, c] = r

        rdma2 = {}
        for c in range(N_CHUNKS):
            for p in range(2):
                rdma1[p, c].wait_recv()
                s = sl(p, c)
                out_ref[s, :] = xb_ref[s, :] + comm_ref[0, s, :]
                r = pltpu.make_async_remote_copy(
                    src_ref=out_ref.at[s],
                    dst_ref=comm_ref.at[1, s],
                    send_sem=send_sems.at[1, p, c],
                    recv_sem=recv_sems.at[1, p, c],
                    device_id=partners[p][1],
                    device_id_type=pl.DeviceIdType.MESH,
                )
                r.start()
                rdma2[p, c] = r

        for c in range(N_CHUNKS):
            for p in range(2):
                rdma2[p, c].wait_send()
                rdma2[p, c].wait_recv()
                s = sl(p, c)
                out_ref[s, :] = out_ref[s, :] + comm_ref[1, s, :]

        for c in range(N_CHUNKS):
            for p in range(2):
                rdma1[p, c].wait_send()

    return pl.pallas_call(
        body,
        out_shape=jax.ShapeDtypeStruct((m, n), jnp.bfloat16),
        in_specs=[pl.BlockSpec(memory_space=pltpu.MemorySpace.HBM)],
        out_specs=pl.BlockSpec(memory_space=pltpu.VMEM),
        scratch_shapes=[
            pltpu.VMEM((m, n), jnp.float32),
            pltpu.VMEM((m, n), jnp.bfloat16),
            pltpu.VMEM((2, m, n), jnp.bfloat16),
            pltpu.SemaphoreType.DMA((2, N_CHUNKS)),
            pltpu.SemaphoreType.DMA((2, 2, N_CHUNKS)),
            pltpu.SemaphoreType.DMA((2, 2, N_CHUNKS)),
        ],
        compiler_params=pltpu.CompilerParams(collective_id=0),
    )(pltpu.with_memory_space_constraint(x, pltpu.MemorySpace.HBM))


# device time: 8538 ns/iter; 1.2316x vs baseline; 1.0273x over previous
import os

import jax
import jax.numpy as jnp
from jax import lax
from jax.experimental import pallas as pl
from jax.experimental.pallas import tpu as pltpu

N_CHUNKS = int(os.environ.get("KCHUNKS", "4"))


def kernel(x):
    m, n = x.shape[2], x.shape[3]
    half = m // 2
    rows = half // N_CHUNKS

    def body(x_hbm, out_ref, xf_ref, xb_ref, acc_ref, comm_ref,
             in_sems, send_sems, recv_sems):
        my_x = lax.axis_index("x")
        my_y = lax.axis_index("y")
        x_nbr = (1 - my_x, my_y)
        y_nbr = (my_x, 1 - my_y)
        partners = ((x_nbr, y_nbr), (y_nbr, x_nbr))

        def sl(p, c):
            return pl.ds(p * half + c * rows, rows)

        in_copies = {}
        for c in range(N_CHUNKS):
            for p in range(2):
                ic = pltpu.make_async_copy(
                    x_hbm.at[0, 0, sl(p, c)], xf_ref.at[sl(p, c)],
                    in_sems.at[p, c],
                )
                ic.start()
                in_copies[p, c] = ic

        barrier_sem = pltpu.get_barrier_semaphore()
        for nbr in (x_nbr, y_nbr):
            pl.semaphore_signal(
                barrier_sem, inc=1,
                device_id=nbr, device_id_type=pl.DeviceIdType.MESH,
            )
        pl.semaphore_wait(barrier_sem, 2)

        rdma1 = {}
        for c in range(N_CHUNKS):
            for p in range(2):
                in_copies[p, c].wait()
                s = sl(p, c)
                xb_ref[s, :] = xf_ref[s, :].astype(jnp.bfloat16)
                r = pltpu.make_async_remote_copy(
                    src_ref=xb_ref.at[s],
                    dst_ref=comm_ref.at[0, s],
                    send_sem=send_sems.at[0, p, c],
                    recv_sem=recv_sems.at[0, p, c],
                    device_id=partners[p][0],
                    device_id_type=pl.DeviceIdType.MESH,
                )
                r.start()
                rdma1[p, c] = r

        rdma2 = {}
        for c in range(N_CHUNKS):
            for p in range(2):
                rdma1[p, c].wait_recv()
                s = sl(p, c)
                acc_ref[s, :] = xb_ref[s, :] + comm_ref[0, s, :]
                r = pltpu.make_async_remote_copy(
                    src_ref=acc_ref.at[s],
                    dst_ref=comm_ref.at[1, s],
                    send_sem=send_sems.at[1, p, c],
                    recv_sem=recv_sems.at[1, p, c],
                    device_id=partners[p][1],
                    device_id_type=pl.DeviceIdType.MESH,
                )
                r.start()
                rdma2[p, c] = r

        for c in range(N_CHUNKS):
            for p in range(2):
                rdma2[p, c].wait_recv()
                s = sl(p, c)
                out_ref[s, :] = acc_ref[s, :] + comm_ref[1, s, :]

        for c in range(N_CHUNKS):
            for p in range(2):
                rdma1[p, c].wait_send()
                rdma2[p, c].wait_send()

    return pl.pallas_call(
        body,
        out_shape=jax.ShapeDtypeStruct((m, n), jnp.bfloat16),
        in_specs=[pl.BlockSpec(memory_space=pltpu.MemorySpace.HBM)],
        out_specs=pl.BlockSpec(memory_space=pltpu.VMEM),
        scratch_shapes=[
            pltpu.VMEM((m, n), jnp.float32),
            pltpu.VMEM((m, n), jnp.bfloat16),
            pltpu.VMEM((m, n), jnp.bfloat16),
            pltpu.VMEM((2, m, n), jnp.bfloat16),
            pltpu.SemaphoreType.DMA((2, N_CHUNKS)),
            pltpu.SemaphoreType.DMA((2, 2, N_CHUNKS)),
            pltpu.SemaphoreType.DMA((2, 2, N_CHUNKS)),
        ],
        compiler_params=pltpu.CompilerParams(collective_id=0),
    )(pltpu.with_memory_space_constraint(x, pltpu.MemorySpace.HBM))
